# Initial kernel scaffold; baseline (speedup 1.0000x reference)
import functools

import jax
import jax.numpy as jnp
from jax import lax
from jax.experimental import pallas as pl
from jax.experimental.pallas import tpu as pltpu
from jax.experimental.pallas import tpu_sc as plsc

_PROB = 0.5

_B, _S, _C, _T = 8, 4, 2, 160000
_ROWS = _B * _S * _C
_NC, _NS = 2, 16
_NW = _NC * _NS
_ROWS_PER_W = _ROWS // _NW
_CHUNK = 80000
_NCHUNK = _T // _CHUNK


def _sc_body(x_hbm, rword_hbm, out_hbm, rv, buf):
    wid = lax.axis_index("s") * _NC + lax.axis_index("c")
    pltpu.sync_copy(rword_hbm, rv)
    rword = jnp.max(rv[...])
    for k in range(_ROWS_PER_W):
        i = wid * _ROWS_PER_W + k
        bit = lax.bitwise_and(lax.shift_right_logical(rword, lax.shift_right_logical(i, 1)), 1)
        j = lax.bitwise_xor(i, bit)
        for c in range(_NCHUNK):
            off = c * _CHUNK
            pltpu.sync_copy(x_hbm.at[pl.ds(i, 1), pl.ds(off, _CHUNK)], buf)
            pltpu.sync_copy(buf, out_hbm.at[pl.ds(j, 1), pl.ds(off, _CHUNK)])


_sc_call = functools.partial(
    pl.kernel,
    mesh=plsc.VectorSubcoreMesh(core_axis_name="c", subcore_axis_name="s"),
    out_type=jax.ShapeDtypeStruct((_ROWS, _T), jnp.float32),
    scratch_types=[
        pltpu.VMEM((16,), jnp.int32),
        pltpu.VMEM((1, _CHUNK), jnp.float32),
    ],
)(_sc_body)


def kernel(signals):
    B, S, C, T = signals.shape
    x = signals.reshape(B * S * C, T)
    rkey = jax.random.fold_in(jax.random.key(0), 42)
    r = jax.random.bernoulli(rkey, _PROB, shape=(B, S)).astype(jnp.int32)
    rflat = r.reshape(-1)
    rword = jnp.sum(rflat << jnp.arange(B * S, dtype=jnp.int32)).astype(jnp.int32)
    rvec = jnp.full((16,), rword, dtype=jnp.int32)
    out = _sc_call(x, rvec)
    return out.reshape(B, S, C, T)


# speedup vs baseline: 138.9171x; 138.9171x over previous
import functools

import jax
import jax.numpy as jnp
from jax import lax
from jax.experimental import pallas as pl
from jax.experimental.pallas import tpu as pltpu
from jax.experimental.pallas import tpu_sc as plsc

_PROB = 0.5

_B, _S, _C, _T = 8, 4, 2, 160000
_ROWS = _B * _S * _C
_NC, _NS = 2, 16
_NW = _NC * _NS
_KK = 25
_CH = _KK * 128
_NCH = _T // _CH
_K = 10
_G = _C * _NCH // _K


def _sc_body(xc_hbm, idx_hbm, out_hbm, idxv, buf, si0, si1, so0, so1):
    wid = lax.axis_index("s") * _NC + lax.axis_index("c")
    pltpu.sync_copy(idx_hbm.at[wid], idxv)

    isem = (si0, si1)
    osem = (so0, so1)

    def gather(g, p):
        return pltpu.async_copy(xc_hbm.at[idxv.at[g]], buf.at[p], isem[p])

    def store(g, p):
        j = _C * wid + (g // (_NCH // _K))
        c0 = (g % (_NCH // _K)) * _K
        return pltpu.async_copy(
            buf.at[p], out_hbm.at[pl.ds(j * _NCH + c0, _K)], osem[p])

    in_h = [None, None]
    out_h = [None, None]
    in_h[0] = gather(0, 0)
    for g in range(_G):
        p = g & 1
        in_h[p].wait()
        if g + 1 < _G:
            if out_h[1 - p] is not None:
                out_h[1 - p].wait()
            in_h[1 - p] = gather(g + 1, 1 - p)
        out_h[p] = store(g, p)
    out_h[(_G - 1) & 1].wait()


_sc_call = functools.partial(
    pl.kernel,
    mesh=plsc.VectorSubcoreMesh(core_axis_name="c", subcore_axis_name="s"),
    out_type=jax.ShapeDtypeStruct((_ROWS * _NCH, _KK, 128), jnp.float32),
    scratch_types=[
        pltpu.VMEM((_G, _K), jnp.int32),
        pltpu.VMEM((2, _K, _KK, 128), jnp.float32),
        pltpu.SemaphoreType.DMA,
        pltpu.SemaphoreType.DMA,
        pltpu.SemaphoreType.DMA,
        pltpu.SemaphoreType.DMA,
    ],
)(_sc_body)


def _chunk_indices():
    rkey = jax.random.fold_in(jax.random.key(0), 42)
    r = jax.random.bernoulli(rkey, _PROB, shape=(_B, _S)).astype(jnp.int32)
    bit = r.reshape(_NW)
    w = jnp.arange(_NW)[:, None, None]
    g = jnp.arange(_G)[None, :, None]
    m = jnp.arange(_K)[None, None, :]
    j = _C * w + g // (_NCH // _K)
    i = j ^ bit[:, None, None]
    c0 = (g % (_NCH // _K)) * _K
    return (i * _NCH + c0 + m).astype(jnp.int32)


def kernel(signals):
    B, S, C, T = signals.shape
    xc = signals.reshape(B * S * C * _NCH, _KK, 128)
    out = _sc_call(xc, _chunk_indices())
    return out.reshape(B, S, C, T)


# speedup vs baseline: 748.4954x; 5.3881x over previous
import functools

import jax
import jax.numpy as jnp
from jax import lax
from jax.experimental import pallas as pl
from jax.experimental.pallas import tpu as pltpu
from jax.experimental.pallas import tpu_sc as plsc

_PROB = 0.5

_B, _S, _C, _T = 8, 4, 2, 160000
_NC, _NS = 2, 16
_NW = _NC * _NS
_L = 32000
_G = _T // _L
_V = 16


def _sc_body(x_hbm, rvec_hbm, out_hbm, rv, buf, si0, si1, so0, so1):
    wid = lax.axis_index("s") * _NC + lax.axis_index("c")
    b = wid // _S
    s = wid % _S
    pltpu.sync_copy(rvec_hbm, rv)
    rword = rv[...]
    bit = lax.shift_right_logical(rword, wid) & 1
    swap = bit == 1

    isem = (si0, si1)
    osem = (so0, so1)

    def gather(g, p):
        return pltpu.async_copy(
            x_hbm.at[b, s, :, pl.ds(g * _L, _L)], buf.at[p], isem[p])

    def store(g, p):
        return pltpu.async_copy(
            buf.at[p], out_hbm.at[b, s, :, pl.ds(g * _L, _L)], osem[p])

    def swap_rows(p):
        def body(t, _):
            o = t * _V
            v0 = buf[p, 0, pl.ds(o, _V)]
            v1 = buf[p, 1, pl.ds(o, _V)]
            buf[p, 0, pl.ds(o, _V)] = jnp.where(swap, v1, v0)
            buf[p, 1, pl.ds(o, _V)] = jnp.where(swap, v0, v1)
            return _
        lax.fori_loop(0, _L // _V, body, None, unroll=8)

    in_h = [None, None]
    out_h = [None, None]
    in_h[0] = gather(0, 0)
    for g in range(_G):
        p = g & 1
        in_h[p].wait()
        if g + 1 < _G:
            if out_h[1 - p] is not None:
                out_h[1 - p].wait()
            in_h[1 - p] = gather(g + 1, 1 - p)
        swap_rows(p)
        out_h[p] = store(g, p)
    out_h[(_G - 1) & 1].wait()


_sc_call = functools.partial(
    pl.kernel,
    mesh=plsc.VectorSubcoreMesh(core_axis_name="c", subcore_axis_name="s"),
    out_type=jax.ShapeDtypeStruct((_B, _S, _C, _T), jnp.float32),
    scratch_types=[
        pltpu.VMEM((_V,), jnp.int32),
        pltpu.VMEM((2, _C, _L), jnp.float32),
        pltpu.SemaphoreType.DMA,
        pltpu.SemaphoreType.DMA,
        pltpu.SemaphoreType.DMA,
        pltpu.SemaphoreType.DMA,
    ],
)(_sc_body)


def _mask_vec():
    rkey = jax.random.fold_in(jax.random.key(0), 42)
    r = jax.random.bernoulli(rkey, _PROB, shape=(_B, _S)).astype(jnp.int32)
    bits = r.reshape(_NW)
    word = jnp.sum(bits << jnp.arange(_NW, dtype=jnp.int32)).astype(jnp.int32)
    return jnp.full((_V,), word, dtype=jnp.int32)


def kernel(signals):
    return _sc_call(signals, _mask_vec())
